# baseline (speedup 1.0000x reference)
import functools
import math

import jax
import jax.numpy as jnp
from jax import lax
from jax.experimental import pallas as pl
from jax.experimental.pallas import tpu as pltpu
from jax.experimental.pallas import tpu_sc as plsc

VOCAB = 1000000
D = 64
SCALE = math.sqrt(D)

_info = plsc.get_sparse_core_info()
NC, NS, L = _info.num_cores, _info.num_subcores, _info.num_lanes
NW = NC * NS

CHUNK = 128
B_TOTAL = 4096 * 200
B_PER_W = B_TOTAL // NW
N_CHUNKS = B_PER_W // CHUNK


def _sc_kernel(idx_hbm, table_hbm, out_hbm, idx_v, rows_v, gsem):
    wid = lax.axis_index("s") * NC + lax.axis_index("c")
    base = wid * B_PER_W

    pltpu.sync_copy(idx_hbm.at[wid], idx_v)

    def chunk_body(j, _):
        pltpu.async_copy(table_hbm.at[idx_v.at[j]], rows_v, gsem).wait()

        def scale_row(r, _):
            for c in range(D // L):
                sl = pl.ds(c * L, L)
                rows_v[r, sl] = rows_v[r, sl] * SCALE
            return 0

        lax.fori_loop(0, CHUNK, scale_row, 0)

        pltpu.sync_copy(rows_v, out_hbm.at[pl.ds(base + j * CHUNK, CHUNK)])
        return 0

    lax.fori_loop(0, N_CHUNKS, chunk_body, 0)


@jax.jit
def kernel(x, table):
    idx = x.reshape(NW, N_CHUNKS, CHUNK)
    mesh = plsc.VectorSubcoreMesh(core_axis_name="c", subcore_axis_name="s")
    out = pl.kernel(
        _sc_kernel,
        mesh=mesh,
        compiler_params=pltpu.CompilerParams(use_tc_tiling_on_sc=False),
        out_type=jax.ShapeDtypeStruct((B_TOTAL, D), jnp.float32),
        scratch_types=[
            pltpu.VMEM((N_CHUNKS, CHUNK), jnp.int32),
            pltpu.VMEM((CHUNK, D), jnp.float32),
            pltpu.SemaphoreType.DMA,
        ],
    )(idx, table)
    return out.reshape(4096, 200, D)


# speedup vs baseline: 1.2105x; 1.2105x over previous
import functools
import math

import jax
import jax.numpy as jnp
from jax import lax
from jax.experimental import pallas as pl
from jax.experimental.pallas import tpu as pltpu
from jax.experimental.pallas import tpu_sc as plsc

VOCAB = 1000000
D = 64
SCALE = math.sqrt(D)

_info = plsc.get_sparse_core_info()
NC, NS, L = _info.num_cores, _info.num_subcores, _info.num_lanes
NW = NC * NS

CHUNK = 128
B_TOTAL = 4096 * 200
B_PER_W = B_TOTAL // NW
N_CHUNKS = B_PER_W // CHUNK
NBUF = 4
N_GROUPS = N_CHUNKS // NBUF
UNROLL = 4


def _sc_kernel(idx_hbm, table_hbm, out_hbm, *scratch):
    idx_v = scratch[0]
    gbufs = scratch[1:1 + NBUF]
    sbufs = scratch[1 + NBUF:1 + 2 * NBUF]
    gsems = scratch[1 + 2 * NBUF:1 + 3 * NBUF]
    ssems = scratch[1 + 3 * NBUF:1 + 4 * NBUF]

    wid = lax.axis_index("s") * NC + lax.axis_index("c")
    base = wid * B_PER_W

    pltpu.sync_copy(idx_hbm.at[wid], idx_v)

    def gather_start(j, b):
        pltpu.async_copy(table_hbm.at[idx_v.at[j]], gbufs[b], gsems[b])

    def gather_wait(b):
        pltpu.make_async_copy(table_hbm.at[idx_v.at[0]], gbufs[b], gsems[b]).wait()

    def store_start(j, b):
        pltpu.async_copy(sbufs[b], out_hbm.at[pl.ds(base + j * CHUNK, CHUNK)],
                         ssems[b])

    def store_wait(b):
        pltpu.make_async_copy(sbufs[b], out_hbm.at[pl.ds(base, CHUNK)],
                              ssems[b]).wait()

    def scale(b):
        g, s = gbufs[b], sbufs[b]

        def scale_rows(r0, _):
            for u in range(UNROLL):
                for c in range(D // L):
                    sl = pl.ds(c * L, L)
                    s[r0 + u, sl] = g[r0 + u, sl] * SCALE
            return 0

        lax.fori_loop(0, CHUNK // UNROLL, lambda i, _: scale_rows(i * UNROLL, _),
                      0, unroll=False)

    for b in range(NBUF):
        gather_start(b, b)

    def group_body(g, _):
        for b in range(NBUF):
            j = g * NBUF + b
            gather_wait(b)

            @pl.when(g > 0)
            def _():
                store_wait(b)

            scale(b)
            store_start(j, b)

            @pl.when(g < N_GROUPS - 1)
            def _():
                gather_start(j + NBUF, b)
        return 0

    lax.fori_loop(0, N_GROUPS, group_body, 0)

    for b in range(NBUF):
        store_wait(b)


@jax.jit
def kernel(x, table):
    idx = x.reshape(NW, N_CHUNKS, CHUNK)
    mesh = plsc.VectorSubcoreMesh(core_axis_name="c", subcore_axis_name="s")
    scratch = (
        [pltpu.VMEM((N_CHUNKS, CHUNK), jnp.int32)]
        + [pltpu.VMEM((CHUNK, D), jnp.float32) for _ in range(2 * NBUF)]
        + [pltpu.SemaphoreType.DMA for _ in range(2 * NBUF)]
    )
    out = pl.kernel(
        _sc_kernel,
        mesh=mesh,
        compiler_params=pltpu.CompilerParams(use_tc_tiling_on_sc=False),
        out_type=jax.ShapeDtypeStruct((B_TOTAL, D), jnp.float32),
        scratch_types=scratch,
    )(idx, table)
    return out.reshape(4096, 200, D)
